# Initial kernel scaffold; baseline (speedup 1.0000x reference)
import jax
import jax.numpy as jnp
from jax.experimental import pallas as pl


def kernel(x, edge_index, W1, b1, gamma, beta, alpha, W2, b2):
    raise NotImplementedError("write your pallas kernel here")


# speedup vs baseline: 7.2113x; 7.2113x over previous
import functools

import jax
import jax.numpy as jnp
from jax import lax
from jax.experimental import pallas as pl
from jax.experimental.pallas import tpu as pltpu
from jax.experimental.pallas import tpu_sc as plsc

N = 10000
D = 256
H = 128
E = 160000
EPS = 1e-5

NS = 16
CH = 128
NCHUNK = 80
EPT = NCHUNK * CH
E_PAD = EPT * NS
ROWS_PT = 632
N_PAD = NS * ROWS_PT

_mesh = plsc.VectorSubcoreMesh(core_axis_name="c", subcore_axis_name="s")


@functools.partial(
    pl.kernel,
    out_type=jax.ShapeDtypeStruct((N_PAD, 16), jnp.float32),
    mesh=_mesh,
    scratch_types=[
        pltpu.VMEM((NCHUNK, CH), jnp.int32),
        pltpu.VMEM((CH, 16), jnp.float32),
        pltpu.VMEM_SHARED((N_PAD, 16), jnp.float32),
    ],
)
def _sc_degree(di_hbm, ones_hbm, zeros_hbm, deg_out, dbuf, obuf, acc):
    c = lax.axis_index("c")
    s = lax.axis_index("s")

    pltpu.sync_copy(zeros_hbm, acc.at[pl.ds(s * ROWS_PT, ROWS_PT)])
    pltpu.sync_copy(ones_hbm, obuf)
    pltpu.sync_copy(di_hbm.at[s], dbuf)

    plsc.subcore_barrier()

    def chunk(j, _):
        pltpu.sync_copy(obuf, acc.at[dbuf.at[j]], add=True)
        return 0

    lax.fori_loop(0, NCHUNK, chunk, 0)

    plsc.subcore_barrier()

    @pl.when(c == 0)
    def _():
        pltpu.sync_copy(acc.at[pl.ds(s * ROWS_PT, ROWS_PT)],
                        deg_out.at[pl.ds(s * ROWS_PT, ROWS_PT)])


@functools.partial(
    pl.kernel,
    out_type=[jax.ShapeDtypeStruct((N_PAD, H), jnp.float32),
              jax.ShapeDtypeStruct((N_PAD, H), jnp.float32)],
    mesh=_mesh,
    scratch_types=[
        pltpu.VMEM((NCHUNK, CH), jnp.int32),
        pltpu.VMEM((NCHUNK, CH), jnp.int32),
        pltpu.VMEM((CH, H), jnp.float32),
        pltpu.VMEM_SHARED((N_PAD, H), jnp.float32),
        pltpu.SemaphoreType.DMA,
    ],
)
def _sc_aggregate(ha, hb, si_hbm, di_hbm, out_a, out_b, sbuf, dbuf, rows, acc,
                  sem):
    c = lax.axis_index("c")
    s = lax.axis_index("s")

    def stage(h_ref):
        pltpu.sync_copy(h_ref.at[pl.ds(s * ROWS_PT, ROWS_PT)],
                        acc.at[pl.ds(s * ROWS_PT, ROWS_PT)])
        pltpu.sync_copy(si_hbm.at[s], sbuf)
        pltpu.sync_copy(di_hbm.at[s], dbuf)

    def edge_loop(h_ref):
        def chunk(j, _):
            pltpu.async_copy(h_ref.at[sbuf.at[j]], rows, sem).wait()
            pltpu.sync_copy(rows, acc.at[dbuf.at[j]], add=True)
            return 0

        lax.fori_loop(0, NCHUNK, chunk, 0)

    def copy_out(out_ref):
        pltpu.sync_copy(acc.at[pl.ds(s * ROWS_PT, ROWS_PT)],
                        out_ref.at[pl.ds(s * ROWS_PT, ROWS_PT)])

    @pl.when(c == 0)
    def _():
        stage(ha)

    @pl.when(c == 1)
    def _():
        stage(hb)

    plsc.subcore_barrier()

    @pl.when(c == 0)
    def _():
        edge_loop(ha)

    @pl.when(c == 1)
    def _():
        edge_loop(hb)

    plsc.subcore_barrier()

    @pl.when(c == 0)
    def _():
        copy_out(out_a)

    @pl.when(c == 1)
    def _():
        copy_out(out_b)


_RP = ROWS_PT
_R = 1000


def _tc_proj1_body(x_ref, w_ref, deg_ref, ha_ref, hb_ref, dinv_ref):
    dinv = lax.rsqrt(deg_ref[:, 0:1] + 1.0)
    h = jnp.dot(x_ref[...], w_ref[...],
                preferred_element_type=jnp.float32) * dinv
    ha_ref[...] = h[:, :H]
    hb_ref[...] = h[:, H:]
    dinv_ref[...] = jnp.broadcast_to(dinv, (_RP, 8))


_tc_proj1 = pl.pallas_call(
    _tc_proj1_body,
    grid=(N_PAD // _RP,),
    in_specs=[
        pl.BlockSpec((_RP, D), lambda i: (i, 0)),
        pl.BlockSpec((D, D), lambda i: (0, 0)),
        pl.BlockSpec((_RP, 16), lambda i: (i, 0)),
    ],
    out_specs=[
        pl.BlockSpec((_RP, H), lambda i: (i, 0)),
        pl.BlockSpec((_RP, H), lambda i: (i, 0)),
        pl.BlockSpec((_RP, 8), lambda i: (i, 0)),
    ],
    out_shape=[
        jax.ShapeDtypeStruct((N_PAD, H), jnp.float32),
        jax.ShapeDtypeStruct((N_PAD, H), jnp.float32),
        jax.ShapeDtypeStruct((N, 8), jnp.float32),
    ],
)


def _tc_norm_proj2_body(aa_ref, ab_ref, dinv_ref, b1_ref, gam_ref, bet_ref,
                        alp_ref, w2_ref, h2a_ref, h2b_ref, ssum, ssq):
    p = pl.program_id(0)
    i = pl.program_id(1)
    t = (jnp.concatenate([aa_ref[...], ab_ref[...]], axis=1)
         * dinv_ref[:, 0:1] + b1_ref[...])

    @pl.when(p == 0)
    def _():
        cs = jnp.sum(t, axis=0, keepdims=True)
        cq = jnp.sum(t * t, axis=0, keepdims=True)

        @pl.when(i == 0)
        def _():
            ssum[...] = cs
            ssq[...] = cq

        @pl.when(i > 0)
        def _():
            ssum[...] += cs
            ssq[...] += cq

    @pl.when(p == 1)
    def _():
        mu = ssum[...] * (1.0 / N)
        msq = ssq[...] * (1.0 / N)
        a = alp_ref[...]
        var = msq - (2.0 * a - a * a) * mu * mu
        g = (t - a * mu) * lax.rsqrt(var + EPS) * gam_ref[...] + bet_ref[...]
        g = jnp.maximum(g, 0.0)
        h2 = jnp.dot(g, w2_ref[...],
                     preferred_element_type=jnp.float32) * dinv_ref[:, 0:1]
        h2a_ref[...] = h2[:, :H]
        h2b_ref[...] = h2[:, H:]


_tc_norm_proj2 = pl.pallas_call(
    _tc_norm_proj2_body,
    grid=(2, N // _R),
    in_specs=[
        pl.BlockSpec((_R, H), lambda p, i: (i, 0)),
        pl.BlockSpec((_R, H), lambda p, i: (i, 0)),
        pl.BlockSpec((_R, 8), lambda p, i: (i, 0)),
        pl.BlockSpec((1, D), lambda p, i: (0, 0)),
        pl.BlockSpec((1, D), lambda p, i: (0, 0)),
        pl.BlockSpec((1, D), lambda p, i: (0, 0)),
        pl.BlockSpec((1, D), lambda p, i: (0, 0)),
        pl.BlockSpec((D, D), lambda p, i: (0, 0)),
    ],
    out_specs=[
        pl.BlockSpec((_R, H), lambda p, i: (i, 0)),
        pl.BlockSpec((_R, H), lambda p, i: (i, 0)),
    ],
    out_shape=[
        jax.ShapeDtypeStruct((N_PAD, H), jnp.float32),
        jax.ShapeDtypeStruct((N_PAD, H), jnp.float32),
    ],
    scratch_shapes=[
        pltpu.VMEM((1, D), jnp.float32),
        pltpu.VMEM((1, D), jnp.float32),
    ],
)


def _tc_final_body(aa_ref, ab_ref, dinv_ref, b2_ref, out_ref):
    out_ref[...] = (jnp.concatenate([aa_ref[...], ab_ref[...]], axis=1)
                    * dinv_ref[:, 0:1] + b2_ref[...])


_tc_final = pl.pallas_call(
    _tc_final_body,
    grid=(N // _R,),
    in_specs=[
        pl.BlockSpec((_R, H), lambda i: (i, 0)),
        pl.BlockSpec((_R, H), lambda i: (i, 0)),
        pl.BlockSpec((_R, 8), lambda i: (i, 0)),
        pl.BlockSpec((1, D), lambda i: (0, 0)),
    ],
    out_specs=pl.BlockSpec((_R, D), lambda i: (i, 0)),
    out_shape=jax.ShapeDtypeStruct((N, D), jnp.float32),
)


def kernel(x, edge_index, W1, b1, gamma, beta, alpha, W2, b2):
    src = edge_index[0]
    dst = edge_index[1]
    pad = E_PAD - E
    src_p = jnp.concatenate(
        [src, jnp.zeros((pad,), jnp.int32)]).reshape(NS, NCHUNK, CH)
    dst_p = jnp.concatenate(
        [dst, jnp.full((pad,), N, jnp.int32)]).reshape(NS, NCHUNK, CH)

    b1r = b1.reshape(1, D)
    b2r = b2.reshape(1, D)
    gammar = gamma.reshape(1, D)
    betar = beta.reshape(1, D)
    alphar = alpha.reshape(1, D)

    deg = _sc_degree(dst_p, jnp.ones((CH, 16), jnp.float32),
                     jnp.zeros((ROWS_PT, 16), jnp.float32))
    ha, hb, dinv = _tc_proj1(x, W1, deg)
    agg1a, agg1b = _sc_aggregate(ha, hb, src_p, dst_p)
    h2a, h2b = _tc_norm_proj2(agg1a, agg1b, dinv, b1r, gammar, betar, alphar,
                              W2)
    agg2a, agg2b = _sc_aggregate(h2a, h2b, src_p, dst_p)
    return _tc_final(agg2a, agg2b, dinv, b2r)
